# Initial kernel scaffold; baseline (speedup 1.0000x reference)
import jax
import jax.numpy as jnp
from jax.experimental import pallas as pl


def kernel(feats, labels, domain_ids):
    raise NotImplementedError("write your pallas kernel here")


# speedup vs baseline: 2.2044x; 2.2044x over previous
import functools

import jax
import jax.numpy as jnp
from jax import lax
from jax.experimental import pallas as pl
from jax.experimental.pallas import tpu as pltpu
from jax.experimental.pallas import tpu_sc as plsc

_C = 256
_D = 32
_M = 4
_N = 1000000
_MOM = 0.9

_NC = 2
_NS = 16
_L = 16
_NW = _NC * _NS
_RPW = _N // _NW
_CHUNK = 625
_NCHUNK = _RPW // _CHUNK
_CHUNK_PAD = 640
_SEG = _M * _C
_SLOT = 48
_ACC = _SEG * _SLOT


def _sc_body(feats_hbm, lab_hbm, dom_hbm, out_hbm, feat_v, lab_v, dom_v,
             seg_v, acc_v):
    wid = lax.axis_index("s") * _NC + lax.axis_index("c")
    iota = lax.iota(jnp.int32, _L)
    zf = jnp.zeros((_L,), jnp.float32)
    cntvec = jnp.where(iota == 0, 1.0, 0.0).astype(jnp.float32)

    def zero_body(i, _):
        plsc.store_scatter(acc_v, [i * _L + iota], zf)
        return ()

    lax.fori_loop(0, _ACC // _L, zero_body, (), unroll=8)

    def chunk_body(c, _):
        g = wid * _NCHUNK + c
        r0 = (wid * _RPW + c * _CHUNK) * _D
        pltpu.sync_copy(feats_hbm.at[pl.ds(r0, _CHUNK * _D)], feat_v)
        pltpu.sync_copy(lab_hbm.at[g], lab_v)
        pltpu.sync_copy(dom_hbm.at[g], dom_v)

        def seg_body(t, _):
            lidx = t * _L + iota
            labv = plsc.load_gather(lab_v, [lidx])
            domv = plsc.load_gather(dom_v, [lidx])
            plsc.store_scatter(seg_v, [lidx], (domv * _C + labv) * _SLOT)
            return ()

        lax.fori_loop(0, _CHUNK_PAD // _L, seg_body, (), unroll=4)

        def row_body(j, _):
            jb = jnp.full((_L,), 0, jnp.int32) + j
            sbase = plsc.load_gather(seg_v, [jb])
            fi = j * _D + iota
            f0 = plsc.load_gather(feat_v, [fi])
            f1 = plsc.load_gather(feat_v, [fi + _L])
            plsc.addupdate_scatter(acc_v, [sbase + iota], f0)
            plsc.addupdate_scatter(acc_v, [sbase + (_L + iota)], f1)
            plsc.addupdate_scatter(acc_v, [sbase + (2 * _L + iota)], cntvec)
            return ()

        lax.fori_loop(0, _CHUNK, row_body, (), unroll=4)
        return ()

    lax.fori_loop(0, _NCHUNK, chunk_body, ())
    pltpu.sync_copy(acc_v, out_hbm.at[wid])


_sc_partials = functools.partial(
    pl.kernel,
    out_type=jax.ShapeDtypeStruct((_NW, _ACC), jnp.float32),
    mesh=plsc.VectorSubcoreMesh(core_axis_name="c", subcore_axis_name="s"),
    compiler_params=pltpu.CompilerParams(needs_layout_passes=False),
    scratch_types=[
        pltpu.VMEM((_CHUNK * _D,), jnp.float32),
        pltpu.VMEM((_CHUNK_PAD,), jnp.int32),
        pltpu.VMEM((_CHUNK_PAD,), jnp.int32),
        pltpu.VMEM((_CHUNK_PAD,), jnp.int32),
        pltpu.VMEM((_ACC,), jnp.float32),
    ],
)(_sc_body)


_BLK = 8000
_NBLK = _N // _BLK


def _s2_body(feat_ref, dom_ref, out_ref):
    i = pl.program_id(0)
    f = feat_ref[...]
    dom = dom_ref[...]

    @pl.when(i == 0)
    def _():
        out_ref[...] = jnp.zeros_like(out_ref)

    masked = jnp.concatenate(
        [f * (dom == d).astype(jnp.float32) for d in range(_M)], axis=-1)
    part = lax.dot_general(
        f, masked, (((0,), (0,)), ((), ())),
        preferred_element_type=jnp.float32,
        precision=lax.Precision.DEFAULT,
    )
    out_ref[...] += part


def _s2_call(feats, dom2):
    return pl.pallas_call(
        _s2_body,
        grid=(_NBLK,),
        in_specs=[
            pl.BlockSpec((_BLK, _D), lambda i: (i, 0)),
            pl.BlockSpec((_BLK, 1), lambda i: (i, 0)),
        ],
        out_specs=pl.BlockSpec((_D, _M * _D), lambda i: (0, 0)),
        out_shape=jax.ShapeDtypeStruct((_D, _M * _D), jnp.float32),
        compiler_params=pltpu.CompilerParams(
            dimension_semantics=("arbitrary",),
            fuse_transposed_lhs_in_matmul=True,
        ),
    )(feats, dom2)


def _final_body(psum_ref, s2_ref, out_ref):
    red = jnp.sum(psum_ref[...], axis=0)
    sums = red[:, :_D].reshape(_M, _C, _D)
    cnts = red[:, _D].reshape(_M, _C)
    s2 = s2_ref[...]

    present = cnts > 0.0
    mu_dc = jnp.where(present[..., None],
                      sums / jnp.maximum(cnts, 1.0)[..., None], 0.0)
    anchors_dc = jnp.where(present[..., None], (1.0 - _MOM) * mu_dc, 0.0)

    anchor_global = jnp.zeros((_C, _D), jnp.float32)
    for d in range(_M):
        upd = _MOM * anchor_global + (1.0 - _MOM) * mu_dc[d]
        anchor_global = jnp.where(present[d][:, None], upd, anchor_global)
    counts_global = cnts.sum(axis=0)

    total = sums.sum(axis=0).sum(axis=0)
    nf = jnp.float32(_N)
    mu = total / nf
    s2_tot = (s2[:, 0:_D] + s2[:, _D:2 * _D]
              + s2[:, 2 * _D:3 * _D] + s2[:, 3 * _D:4 * _D])
    cov = (s2_tot - nf * mu[:, None] * mu[None, :]) / (nf + 1e-06)
    gm = (1.0 - _MOM) * mu
    r_i = lax.broadcasted_iota(jnp.int32, (_D, _D), 0)
    c_i = lax.broadcasted_iota(jnp.int32, (_D, _D), 1)
    eye = (r_i == c_i).astype(jnp.float32)
    gc = _MOM * eye + (1.0 - _MOM) * cov

    valid = present & (counts_global > 0.0)[None, :]
    per = ((anchors_dc - anchor_global[None]) ** 2).mean(axis=-1)
    nvalid = valid.sum()
    caa = jnp.where(
        nvalid > 0,
        jnp.where(valid, per, 0.0).sum()
        / jnp.maximum(nvalid, 1).astype(jnp.float32),
        0.0,
    )

    loss_s = jnp.float32(0.0)
    vs = jnp.float32(0.0)
    for d in range(_M):
        n_d = cnts[d].sum()
        sum_d = sums[d].sum(axis=0)
        mu_d = sum_d / jnp.maximum(n_d, 1.0)
        s2_d = s2[:, d * _D:(d + 1) * _D]
        cov_d = (s2_d - n_d * mu_d[:, None] * mu_d[None, :]) / (n_d + 1e-06)
        term = ((mu_d - gm) ** 2).mean() + ((cov_d - gc) ** 2).mean()
        ok = (n_d > 0).astype(jnp.float32)
        loss_s = loss_s + ok * term
        vs = vs + ok
    stats = jnp.where(vs > 0, loss_s / jnp.maximum(vs, 1.0), 0.0)

    out_ref[...] = jnp.full((1, 1), caa + stats, jnp.float32)


def _final_call(psums, s2):
    return pl.pallas_call(
        _final_body,
        out_shape=jax.ShapeDtypeStruct((1, 1), jnp.float32),
    )(psums, s2)


def kernel(feats, labels, domain_ids):
    feats_flat = feats.reshape(_N * _D)
    lab2 = jnp.pad(labels.reshape(_NW * _NCHUNK, _CHUNK),
                   ((0, 0), (0, _CHUNK_PAD - _CHUNK)))
    dom2 = jnp.pad(domain_ids.reshape(_NW * _NCHUNK, _CHUNK),
                   ((0, 0), (0, _CHUNK_PAD - _CHUNK)))
    psums = _sc_partials(feats_flat, lab2, dom2)
    s2 = _s2_call(feats, domain_ids.reshape(_N, 1))
    out = _final_call(psums.reshape(_NW, _SEG, _SLOT), s2)
    return out.reshape(())


# speedup vs baseline: 2.4080x; 1.0924x over previous
import functools

import jax
import jax.numpy as jnp
from jax import lax
from jax.experimental import pallas as pl
from jax.experimental.pallas import tpu as pltpu
from jax.experimental.pallas import tpu_sc as plsc

_C = 256
_D = 32
_M = 4
_N = 1000000
_MOM = 0.9

_NC = 2
_NS = 16
_L = 16
_NW = _NC * _NS
_SEG = _M * _C
_SLOT = 48

_CH = 2000
_GPC = _CH // _L
_NCHTOT = _N // _CH
_CPT = (_NCHTOT + _NW - 1) // _NW


def _sc_body(ft_hbm, lab_hbm, dom_hbm, out_hbm, feat_v, lab_v, dom_v,
             acc_v):
    wid = lax.axis_index("s") * _NC + lax.axis_index("c")
    iota = lax.iota(jnp.int32, _L)
    col0 = iota
    col1 = iota + _L
    zf = jnp.zeros((_L,), jnp.float32)
    zi = jnp.zeros((_L,), jnp.int32)
    cntvec = jnp.where(iota == 0, 1.0, 0.0).astype(jnp.float32)

    def zero_body(s, _):
        acc_v[pl.ds(s * _L, _L)] = zf
        return ()

    lax.fori_loop(0, _SEG * _SLOT // _L, zero_body, (), unroll=8)

    def gbody(g, _):
        labv = lab_v[pl.ds(g * _L, _L)]
        domv = dom_v[pl.ds(g * _L, _L)]
        segv = (domv * _C + labv) * _SLOT
        for j in range(_L):
            segb = segv.at[jnp.full((_L,), j, jnp.int32)].get(
                mode="promise_in_bounds")
            rb = zi + (g * _L + j) * _D
            f0 = plsc.load_gather(feat_v, [rb + col0])
            f1 = plsc.load_gather(feat_v, [rb + col1])
            plsc.addupdate_scatter(acc_v, [segb + col0], f0)
            plsc.addupdate_scatter(acc_v, [segb + col1], f1)
            plsc.addupdate_scatter(acc_v, [segb + 2 * _L], cntvec)
        return ()

    def cbody(cl, _):
        cid = wid + _NW * cl

        @pl.when(cid < _NCHTOT)
        def _():
            c0 = cid * _CH
            pltpu.sync_copy(ft_hbm.at[pl.ds(c0 * _D, _CH * _D)], feat_v)
            pltpu.sync_copy(lab_hbm.at[pl.ds(c0, _CH)], lab_v)
            pltpu.sync_copy(dom_hbm.at[pl.ds(c0, _CH)], dom_v)
            lax.fori_loop(0, _GPC, gbody, ())

        return ()

    lax.fori_loop(0, _CPT, cbody, ())
    pltpu.sync_copy(acc_v, out_hbm.at[wid])


_sc_partials = functools.partial(
    pl.kernel,
    out_type=jax.ShapeDtypeStruct((_NW, _SEG * _SLOT), jnp.float32),
    mesh=plsc.VectorSubcoreMesh(core_axis_name="c", subcore_axis_name="s"),
    compiler_params=pltpu.CompilerParams(needs_layout_passes=False),
    scratch_types=[
        pltpu.VMEM((_CH * _D,), jnp.float32),
        pltpu.VMEM((_CH,), jnp.int32),
        pltpu.VMEM((_CH,), jnp.int32),
        pltpu.VMEM((_SEG * _SLOT,), jnp.float32),
    ],
)(_sc_body)


_BT = 8000
_NBLK = _N // _BT


def _s2_body(ft_ref, dom_ref, out_ref):
    i = pl.program_id(0)
    ft = ft_ref[...]
    dom = dom_ref[...]
    h1 = (1 - 2 * (dom & 1)).astype(jnp.bfloat16)
    h2 = (1 - 2 * (dom >> 1)).astype(jnp.bfloat16)
    h3 = h1 * h2
    ftb = ft.astype(jnp.bfloat16)
    lhs = jnp.concatenate(
        [ftb, ftb * h1, ftb * h2, ftb * h3], axis=1)
    part = lax.dot_general(
        lhs, ftb, (((0,), (0,)), ((), ())),
        preferred_element_type=jnp.float32,
    )

    @pl.when(i == 0)
    def _():
        out_ref[...] = jnp.zeros_like(out_ref)

    out_ref[...] += part


def _s2_call(f, dom2):
    return pl.pallas_call(
        _s2_body,
        grid=(_NBLK,),
        in_specs=[
            pl.BlockSpec((_BT, _D), lambda i: (i, 0)),
            pl.BlockSpec((_BT, 1), lambda i: (i, 0)),
        ],
        out_specs=pl.BlockSpec((_M * _D, _D), lambda i: (0, 0)),
        out_shape=jax.ShapeDtypeStruct((_M * _D, _D), jnp.float32),
        compiler_params=pltpu.CompilerParams(
            dimension_semantics=("arbitrary",),
        ),
    )(f, dom2)


def _final_body(psum_ref, t_ref, out_ref):
    p3 = psum_ref[...].reshape(_NW, _SEG, _SLOT)
    red = jnp.sum(p3, axis=0)
    sums = red[:, :_D].reshape(_M, _C, _D)
    cnts = red[:, _D].reshape(_M, _C)
    t = t_ref[...]
    t0 = t[0:_D, :]
    t1 = t[_D:2 * _D, :]
    t2 = t[2 * _D:3 * _D, :]
    t3 = t[3 * _D:4 * _D, :]

    present = cnts > 0.0
    mu_dc = jnp.where(present[..., None],
                      sums / jnp.maximum(cnts, 1.0)[..., None], 0.0)
    anchors_dc = jnp.where(present[..., None], (1.0 - _MOM) * mu_dc, 0.0)

    anchor_global = jnp.zeros((_C, _D), jnp.float32)
    for d in range(_M):
        upd = _MOM * anchor_global + (1.0 - _MOM) * mu_dc[d]
        anchor_global = jnp.where(present[d][:, None], upd, anchor_global)
    counts_global = cnts.sum(axis=0)

    total = sums.sum(axis=0).sum(axis=0)
    nf = jnp.float32(_N)
    mu = total / nf
    cov = (t0 - nf * mu[:, None] * mu[None, :]) / (nf + 1e-06)
    gm = (1.0 - _MOM) * mu
    r_i = lax.broadcasted_iota(jnp.int32, (_D, _D), 0)
    c_i = lax.broadcasted_iota(jnp.int32, (_D, _D), 1)
    eye = (r_i == c_i).astype(jnp.float32)
    gc = _MOM * eye + (1.0 - _MOM) * cov

    valid = present & (counts_global > 0.0)[None, :]
    per = ((anchors_dc - anchor_global[None]) ** 2).mean(axis=-1)
    nvalid = valid.sum()
    caa = jnp.where(
        nvalid > 0,
        jnp.where(valid, per, 0.0).sum()
        / jnp.maximum(nvalid, 1).astype(jnp.float32),
        0.0,
    )

    loss_s = jnp.float32(0.0)
    vs = jnp.float32(0.0)
    for d in range(_M):
        n_d = cnts[d].sum()
        sum_d = sums[d].sum(axis=0)
        mu_d = sum_d / jnp.maximum(n_d, 1.0)
        s1 = 1.0 - 2.0 * (d & 1)
        s2 = 1.0 - 2.0 * (d >> 1)
        s2_d = 0.25 * (t0 + s1 * t1 + s2 * t2 + (s1 * s2) * t3)
        cov_d = (s2_d - n_d * mu_d[:, None] * mu_d[None, :]) / (n_d + 1e-06)
        term = ((mu_d - gm) ** 2).mean() + ((cov_d - gc) ** 2).mean()
        ok = (n_d > 0).astype(jnp.float32)
        loss_s = loss_s + ok * term
        vs = vs + ok
    stats = jnp.where(vs > 0, loss_s / jnp.maximum(vs, 1.0), 0.0)

    out_ref[...] = jnp.full((1, 1), caa + stats, jnp.float32)


def _final_call(psums, t):
    return pl.pallas_call(
        _final_body,
        out_shape=jax.ShapeDtypeStruct((1, 1), jnp.float32),
    )(psums, t)


def kernel(feats, labels, domain_ids):
    psums = _sc_partials(feats.reshape(_N * _D), labels, domain_ids)
    t = _s2_call(feats, domain_ids.reshape(_N, 1))
    out = _final_call(psums.reshape(_NW * _SEG, _SLOT), t)
    return out.reshape(())


# speedup vs baseline: 2.4105x; 1.0011x over previous
import functools

import jax
import jax.numpy as jnp
from jax import lax
from jax.experimental import pallas as pl
from jax.experimental.pallas import tpu as pltpu
from jax.experimental.pallas import tpu_sc as plsc

_C = 256
_D = 32
_M = 4
_N = 1000000
_MOM = 0.9

_NC = 2
_NS = 16
_L = 16
_NW = _NC * _NS
_SEG = _M * _C
_SLOT = 48

_CH = 2000
_GPC = _CH // _L
_NCHTOT = _N // _CH
_CPT = (_NCHTOT + _NW - 1) // _NW


def _sc_body(ft_hbm, lab_hbm, dom_hbm, out_hbm, feat_v, lab_v, dom_v,
             acc_v):
    wid = lax.axis_index("s") * _NC + lax.axis_index("c")
    iota = lax.iota(jnp.int32, _L)
    col0 = iota
    col1 = iota + _L
    zf = jnp.zeros((_L,), jnp.float32)
    zi = jnp.zeros((_L,), jnp.int32)
    ones = jnp.ones((_L,), jnp.float32)

    def zero_body(s, _):
        acc_v[pl.ds(s * _L, _L)] = zf
        return ()

    lax.fori_loop(0, _SEG * _SLOT // _L, zero_body, (), unroll=8)

    def gbody(g, _):
        labv = lab_v[pl.ds(g * _L, _L)]
        domv = dom_v[pl.ds(g * _L, _L)]
        segv = (domv * _C + labv) * _SLOT
        plsc.addupdate_scatter(acc_v, [segv + 2 * _L], ones)
        for j in range(_L):
            segb = segv.at[jnp.full((_L,), j, jnp.int32)].get(
                mode="promise_in_bounds")
            tok = (g * _L + j) * _D
            f0 = feat_v[pl.ds(tok, _L)]
            f1 = feat_v[pl.ds(tok + _L, _L)]
            plsc.addupdate_scatter(acc_v, [segb + col0], f0)
            plsc.addupdate_scatter(acc_v, [segb + col1], f1)
        return ()

    def cbody(cl, _):
        cid = wid + _NW * cl

        @pl.when(cid < _NCHTOT)
        def _():
            c0 = cid * _CH
            pltpu.sync_copy(ft_hbm.at[pl.ds(c0 * _D, _CH * _D)], feat_v)
            pltpu.sync_copy(lab_hbm.at[pl.ds(c0, _CH)], lab_v)
            pltpu.sync_copy(dom_hbm.at[pl.ds(c0, _CH)], dom_v)
            lax.fori_loop(0, _GPC, gbody, ())

        return ()

    lax.fori_loop(0, _CPT, cbody, ())
    pltpu.sync_copy(acc_v, out_hbm.at[wid])


_sc_partials = functools.partial(
    pl.kernel,
    out_type=jax.ShapeDtypeStruct((_NW, _SEG * _SLOT), jnp.float32),
    mesh=plsc.VectorSubcoreMesh(core_axis_name="c", subcore_axis_name="s"),
    compiler_params=pltpu.CompilerParams(needs_layout_passes=False),
    scratch_types=[
        pltpu.VMEM((_CH * _D,), jnp.float32),
        pltpu.VMEM((_CH,), jnp.int32),
        pltpu.VMEM((_CH,), jnp.int32),
        pltpu.VMEM((_SEG * _SLOT,), jnp.float32),
    ],
)(_sc_body)


_BT = 8000
_NBLK = _N // _BT


def _s2_body(ft_ref, dom_ref, out_ref):
    i = pl.program_id(0)
    ft = ft_ref[...]
    dom = dom_ref[...]
    h1 = (1 - 2 * (dom & 1)).astype(jnp.bfloat16)
    h2 = (1 - 2 * (dom >> 1)).astype(jnp.bfloat16)
    h3 = h1 * h2
    ftb = ft.astype(jnp.bfloat16)
    lhs = jnp.concatenate(
        [ftb, ftb * h1, ftb * h2, ftb * h3], axis=1)
    part = lax.dot_general(
        lhs, ftb, (((0,), (0,)), ((), ())),
        preferred_element_type=jnp.float32,
    )

    @pl.when(i == 0)
    def _():
        out_ref[...] = jnp.zeros_like(out_ref)

    out_ref[...] += part


def _s2_call(f, dom2):
    return pl.pallas_call(
        _s2_body,
        grid=(_NBLK,),
        in_specs=[
            pl.BlockSpec((_BT, _D), lambda i: (i, 0)),
            pl.BlockSpec((_BT, 1), lambda i: (i, 0)),
        ],
        out_specs=pl.BlockSpec((_M * _D, _D), lambda i: (0, 0)),
        out_shape=jax.ShapeDtypeStruct((_M * _D, _D), jnp.float32),
        compiler_params=pltpu.CompilerParams(
            dimension_semantics=("arbitrary",),
        ),
    )(f, dom2)


def _final_body(psum_ref, t_ref, out_ref):
    p3 = psum_ref[...].reshape(_NW, _SEG, _SLOT)
    red = jnp.sum(p3, axis=0)
    sums = red[:, :_D].reshape(_M, _C, _D)
    cnts = red[:, _D].reshape(_M, _C)
    t = t_ref[...]
    t0 = t[0:_D, :]
    t1 = t[_D:2 * _D, :]
    t2 = t[2 * _D:3 * _D, :]
    t3 = t[3 * _D:4 * _D, :]

    present = cnts > 0.0
    mu_dc = jnp.where(present[..., None],
                      sums / jnp.maximum(cnts, 1.0)[..., None], 0.0)
    anchors_dc = jnp.where(present[..., None], (1.0 - _MOM) * mu_dc, 0.0)

    anchor_global = jnp.zeros((_C, _D), jnp.float32)
    for d in range(_M):
        upd = _MOM * anchor_global + (1.0 - _MOM) * mu_dc[d]
        anchor_global = jnp.where(present[d][:, None], upd, anchor_global)
    counts_global = cnts.sum(axis=0)

    total = sums.sum(axis=0).sum(axis=0)
    nf = jnp.float32(_N)
    mu = total / nf
    cov = (t0 - nf * mu[:, None] * mu[None, :]) / (nf + 1e-06)
    gm = (1.0 - _MOM) * mu
    r_i = lax.broadcasted_iota(jnp.int32, (_D, _D), 0)
    c_i = lax.broadcasted_iota(jnp.int32, (_D, _D), 1)
    eye = (r_i == c_i).astype(jnp.float32)
    gc = _MOM * eye + (1.0 - _MOM) * cov

    valid = present & (counts_global > 0.0)[None, :]
    per = ((anchors_dc - anchor_global[None]) ** 2).mean(axis=-1)
    nvalid = valid.sum()
    caa = jnp.where(
        nvalid > 0,
        jnp.where(valid, per, 0.0).sum()
        / jnp.maximum(nvalid, 1).astype(jnp.float32),
        0.0,
    )

    loss_s = jnp.float32(0.0)
    vs = jnp.float32(0.0)
    for d in range(_M):
        n_d = cnts[d].sum()
        sum_d = sums[d].sum(axis=0)
        mu_d = sum_d / jnp.maximum(n_d, 1.0)
        s1 = 1.0 - 2.0 * (d & 1)
        s2 = 1.0 - 2.0 * (d >> 1)
        s2_d = 0.25 * (t0 + s1 * t1 + s2 * t2 + (s1 * s2) * t3)
        cov_d = (s2_d - n_d * mu_d[:, None] * mu_d[None, :]) / (n_d + 1e-06)
        term = ((mu_d - gm) ** 2).mean() + ((cov_d - gc) ** 2).mean()
        ok = (n_d > 0).astype(jnp.float32)
        loss_s = loss_s + ok * term
        vs = vs + ok
    stats = jnp.where(vs > 0, loss_s / jnp.maximum(vs, 1.0), 0.0)

    out_ref[...] = jnp.full((1, 1), caa + stats, jnp.float32)


def _final_call(psums, t):
    return pl.pallas_call(
        _final_body,
        out_shape=jax.ShapeDtypeStruct((1, 1), jnp.float32),
    )(psums, t)


def kernel(feats, labels, domain_ids):
    psums = _sc_partials(feats.reshape(_N * _D), labels, domain_ids)
    t = _s2_call(feats, domain_ids.reshape(_N, 1))
    out = _final_call(psums.reshape(_NW * _SEG, _SLOT), t)
    return out.reshape(())


# speedup vs baseline: 2.4302x; 1.0082x over previous
import functools

import jax
import jax.numpy as jnp
from jax import lax
from jax.experimental import pallas as pl
from jax.experimental.pallas import tpu as pltpu
from jax.experimental.pallas import tpu_sc as plsc

_C = 256
_D = 32
_M = 4
_N = 1000000
_MOM = 0.9

_NC = 2
_NS = 16
_L = 16
_NW = _NC * _NS
_SEG = _M * _C

_CH = 2000
_GPC = _CH // _L
_NCHTOT = _N // _CH
_CPT = (_NCHTOT + _NW - 1) // _NW
_ZR = 128


def _sc_body(ft_hbm, lab_hbm, dom_hbm, sum_hbm, cnt_hbm,
             feat_v, lab_v, dom_v, idx_v, cnt_v, zero_v, shared):
    core = lax.axis_index("c")
    sid = lax.axis_index("s")
    wid = sid * _NC + core
    iota = lax.iota(jnp.int32, _L)
    zf = jnp.zeros((_L,), jnp.float32)
    ones = jnp.ones((_L,), jnp.float32)

    def zcnt(s, _):
        cnt_v[pl.ds(s * _L, _L)] = zf
        return ()

    lax.fori_loop(0, _SEG // _L, zcnt, (), unroll=8)

    @pl.when(sid == 0)
    def _():
        def zrow(r, _):
            zero_v[r, pl.ds(0, _L)] = zf
            zero_v[r, pl.ds(_L, _L)] = zf
            return ()

        lax.fori_loop(0, _ZR, zrow, (), unroll=8)

        def zshared(b, _):
            pltpu.sync_copy(zero_v, shared.at[pl.ds(b * _ZR, _ZR), :])
            return ()

        lax.fori_loop(0, _SEG // _ZR, zshared, ())

    plsc.subcore_barrier()

    def gbody(g, _):
        labv = lab_v[pl.ds(g * _L, _L)]
        domv = dom_v[pl.ds(g * _L, _L)]
        segv = domv * _C + labv
        idx_v[pl.ds(g * _L, _L)] = segv
        plsc.addupdate_scatter(cnt_v, [segv], ones)
        return ()

    def cbody(cl, _):
        cid = wid + _NW * cl

        @pl.when(cid < _NCHTOT)
        def _():
            c0 = cid * _CH
            pltpu.sync_copy(ft_hbm.at[pl.ds(c0, _CH), :], feat_v)
            pltpu.sync_copy(lab_hbm.at[pl.ds(c0, _CH)], lab_v)
            pltpu.sync_copy(dom_hbm.at[pl.ds(c0, _CH)], dom_v)
            lax.fori_loop(0, _GPC, gbody, ())
            pltpu.sync_copy(feat_v, shared.at[idx_v], add=True)

        return ()

    lax.fori_loop(0, _CPT, cbody, ())
    plsc.subcore_barrier()

    @pl.when(sid == 0)
    def _():
        pltpu.sync_copy(shared, sum_hbm.at[core])

    pltpu.sync_copy(cnt_v, cnt_hbm.at[wid])


_sc_partials = functools.partial(
    pl.kernel,
    out_type=(
        jax.ShapeDtypeStruct((_NC, _SEG, _D), jnp.float32),
        jax.ShapeDtypeStruct((_NW, _SEG), jnp.float32),
    ),
    mesh=plsc.VectorSubcoreMesh(core_axis_name="c", subcore_axis_name="s"),
    compiler_params=pltpu.CompilerParams(
        needs_layout_passes=False,
        use_tc_tiling_on_sc=False,
    ),
    scratch_types=[
        pltpu.VMEM((_CH, _D), jnp.float32),
        pltpu.VMEM((_CH,), jnp.int32),
        pltpu.VMEM((_CH,), jnp.int32),
        pltpu.VMEM((_CH,), jnp.int32),
        pltpu.VMEM((_SEG,), jnp.float32),
        pltpu.VMEM((_ZR, _D), jnp.float32),
        pltpu.VMEM_SHARED((_SEG, _D), jnp.float32),
    ],
)(_sc_body)


_BT = 8000
_NBLK = _N // _BT


def _s2_body(ft_ref, dom_ref, out_ref):
    i = pl.program_id(0)
    ft = ft_ref[...]
    dom = dom_ref[...]
    h1 = (1 - 2 * (dom & 1)).astype(jnp.bfloat16)
    h2 = (1 - 2 * (dom >> 1)).astype(jnp.bfloat16)
    h3 = h1 * h2
    ftb = ft.astype(jnp.bfloat16)
    lhs = jnp.concatenate(
        [ftb, ftb * h1, ftb * h2, ftb * h3], axis=1)
    part = lax.dot_general(
        lhs, ftb, (((0,), (0,)), ((), ())),
        preferred_element_type=jnp.float32,
    )

    @pl.when(i == 0)
    def _():
        out_ref[...] = jnp.zeros_like(out_ref)

    out_ref[...] += part


def _s2_call(f, dom2):
    return pl.pallas_call(
        _s2_body,
        grid=(_NBLK,),
        in_specs=[
            pl.BlockSpec((_BT, _D), lambda i: (i, 0)),
            pl.BlockSpec((_BT, 1), lambda i: (i, 0)),
        ],
        out_specs=pl.BlockSpec((_M * _D, _D), lambda i: (0, 0)),
        out_shape=jax.ShapeDtypeStruct((_M * _D, _D), jnp.float32),
        compiler_params=pltpu.CompilerParams(
            dimension_semantics=("arbitrary",),
        ),
    )(f, dom2)


def _final_body(sum_ref, cnt_ref, t_ref, out_ref):
    sums2 = sum_ref[...]
    sums = (sums2[:_SEG, :] + sums2[_SEG:, :]).reshape(_M, _C, _D)
    cnt32 = cnt_ref[...].T
    cnts = jnp.sum(cnt32, axis=1).reshape(_M, _C)
    t = t_ref[...]
    t0 = t[0:_D, :]
    t1 = t[_D:2 * _D, :]
    t2 = t[2 * _D:3 * _D, :]
    t3 = t[3 * _D:4 * _D, :]

    present = cnts > 0.0
    mu_dc = jnp.where(present[..., None],
                      sums / jnp.maximum(cnts, 1.0)[..., None], 0.0)
    anchors_dc = jnp.where(present[..., None], (1.0 - _MOM) * mu_dc, 0.0)

    anchor_global = jnp.zeros((_C, _D), jnp.float32)
    for d in range(_M):
        upd = _MOM * anchor_global + (1.0 - _MOM) * mu_dc[d]
        anchor_global = jnp.where(present[d][:, None], upd, anchor_global)
    counts_global = cnts.sum(axis=0)

    total = sums.sum(axis=0).sum(axis=0)
    nf = jnp.float32(_N)
    mu = total / nf
    cov = (t0 - nf * mu[:, None] * mu[None, :]) / (nf + 1e-06)
    gm = (1.0 - _MOM) * mu
    r_i = lax.broadcasted_iota(jnp.int32, (_D, _D), 0)
    c_i = lax.broadcasted_iota(jnp.int32, (_D, _D), 1)
    eye = (r_i == c_i).astype(jnp.float32)
    gc = _MOM * eye + (1.0 - _MOM) * cov

    valid = present & (counts_global > 0.0)[None, :]
    per = ((anchors_dc - anchor_global[None]) ** 2).mean(axis=-1)
    nvalid = valid.sum()
    caa = jnp.where(
        nvalid > 0,
        jnp.where(valid, per, 0.0).sum()
        / jnp.maximum(nvalid, 1).astype(jnp.float32),
        0.0,
    )

    loss_s = jnp.float32(0.0)
    vs = jnp.float32(0.0)
    for d in range(_M):
        n_d = cnts[d].sum()
        sum_d = sums[d].sum(axis=0)
        mu_d = sum_d / jnp.maximum(n_d, 1.0)
        s1 = 1.0 - 2.0 * (d & 1)
        s2 = 1.0 - 2.0 * (d >> 1)
        s2_d = 0.25 * (t0 + s1 * t1 + s2 * t2 + (s1 * s2) * t3)
        cov_d = (s2_d - n_d * mu_d[:, None] * mu_d[None, :]) / (n_d + 1e-06)
        term = ((mu_d - gm) ** 2).mean() + ((cov_d - gc) ** 2).mean()
        ok = (n_d > 0).astype(jnp.float32)
        loss_s = loss_s + ok * term
        vs = vs + ok
    stats = jnp.where(vs > 0, loss_s / jnp.maximum(vs, 1.0), 0.0)

    out_ref[...] = jnp.full((1, 1), caa + stats, jnp.float32)


def _final_call(sums, cnts, t):
    return pl.pallas_call(
        _final_body,
        out_shape=jax.ShapeDtypeStruct((1, 1), jnp.float32),
    )(sums, cnts, t)


def kernel(feats, labels, domain_ids):
    sums, cnts = _sc_partials(feats, labels, domain_ids)
    t = _s2_call(feats, domain_ids.reshape(_N, 1))
    out = _final_call(sums.reshape(_NC * _SEG, _D), cnts, t)
    return out.reshape(())


# speedup vs baseline: 3.1742x; 1.3062x over previous
import functools

import jax
import jax.numpy as jnp
from jax import lax
from jax.experimental import pallas as pl
from jax.experimental.pallas import tpu as pltpu
from jax.experimental.pallas import tpu_sc as plsc

_C = 256
_D = 32
_M = 4
_N = 1000000
_MOM = 0.9

_NC = 2
_NS = 16
_L = 16
_NW = _NC * _NS
_SEG = _M * _C

_CH = 2000
_GPC = _CH // _L
_NCHTOT = _N // _CH
_CPT = (_NCHTOT + _NW - 1) // _NW
_ZR = 128


def _sc_body(ft_hbm, lab_hbm, dom_hbm, sum_hbm, cnt_hbm,
             feat_v, lab_v, dom_v, idx_v, cnt_v, zero_v, shared, sem):
    core = lax.axis_index("c")
    sid = lax.axis_index("s")
    wid = sid * _NC + core
    iota = lax.iota(jnp.int32, _L)
    zf = jnp.zeros((_L,), jnp.float32)
    ones = jnp.ones((_L,), jnp.float32)

    def zcnt(s, _):
        cnt_v[pl.ds(s * _L, _L)] = zf
        return ()

    lax.fori_loop(0, _SEG // _L, zcnt, (), unroll=8)

    @pl.when(sid == 0)
    def _():
        def zrow(r, _):
            zero_v[r, pl.ds(0, _L)] = zf
            zero_v[r, pl.ds(_L, _L)] = zf
            return ()

        lax.fori_loop(0, _ZR, zrow, (), unroll=8)

        def zshared(b, _):
            pltpu.sync_copy(zero_v, shared.at[pl.ds(b * _ZR, _ZR), :])
            return ()

        lax.fori_loop(0, _SEG // _ZR, zshared, ())

    plsc.subcore_barrier()

    def gbody(g, _):
        labv = lab_v[pl.ds(g * _L, _L)]
        domv = dom_v[pl.ds(g * _L, _L)]
        segv = domv * _C + labv
        idx_v[pl.ds(g * _L, _L)] = segv
        plsc.addupdate_scatter(cnt_v, [segv], ones)
        return ()

    def cbody(cl, _):
        cid = wid + _NW * cl

        @pl.when(cid < _NCHTOT)
        def _():
            c0 = cid * _CH
            pltpu.async_copy(ft_hbm.at[pl.ds(c0, _CH), :], feat_v,
                             sem).wait()
            pltpu.sync_copy(lab_hbm.at[pl.ds(c0, _CH)], lab_v)
            pltpu.sync_copy(dom_hbm.at[pl.ds(c0, _CH)], dom_v)
            lax.fori_loop(0, _GPC, gbody, ())
            pltpu.sync_copy(feat_v, shared.at[idx_v], add=True)

        return ()

    lax.fori_loop(0, _CPT, cbody, ())
    plsc.subcore_barrier()

    @pl.when(sid == 0)
    def _():
        pltpu.sync_copy(shared, sum_hbm.at[core])

    pltpu.sync_copy(cnt_v, cnt_hbm.at[wid])


_sc_partials = functools.partial(
    pl.kernel,
    out_type=(
        jax.ShapeDtypeStruct((_NC, _SEG, _D), jnp.float32),
        jax.ShapeDtypeStruct((_NW, _SEG), jnp.float32),
    ),
    mesh=plsc.VectorSubcoreMesh(core_axis_name="c", subcore_axis_name="s"),
    compiler_params=pltpu.CompilerParams(
        needs_layout_passes=False,
        use_tc_tiling_on_sc=False,
    ),
    scratch_types=[
        pltpu.VMEM((_CH, _D), jnp.float32),
        pltpu.VMEM((_CH,), jnp.int32),
        pltpu.VMEM((_CH,), jnp.int32),
        pltpu.VMEM((_CH,), jnp.int32),
        pltpu.VMEM((_SEG,), jnp.float32),
        pltpu.VMEM((_ZR, _D), jnp.float32),
        pltpu.VMEM_SHARED((_SEG, _D), jnp.float32),
        pltpu.SemaphoreType.DMA,
    ],
)(_sc_body)


_TPR = 128 // _D
_NR = _N // _TPR
_BR = 2000
_NBLK = _NR // _BR


def _s2_body(fv_ref, d4_ref, out_ref):
    i = pl.program_id(0)
    vb = fv_ref[...].astype(jnp.bfloat16)
    d4 = d4_ref[...]
    h1 = (1 - 2 * (d4 & 1)).astype(jnp.bfloat16)
    h2 = (1 - 2 * (d4 >> 1)).astype(jnp.bfloat16)
    h3 = h1 * h2
    r_i = lax.broadcasted_iota(jnp.int32, (_TPR, 128), 0)
    c_i = lax.broadcasted_iota(jnp.int32, (_TPR, 128), 1)
    rep = (c_i // _D == r_i).astype(jnp.bfloat16)
    dn = (((1,), (0,)), ((), ()))
    w1 = vb * lax.dot_general(
        h1, rep, dn, preferred_element_type=jnp.float32).astype(jnp.bfloat16)
    w2 = vb * lax.dot_general(
        h2, rep, dn, preferred_element_type=jnp.float32).astype(jnp.bfloat16)
    w3 = vb * lax.dot_general(
        h3, rep, dn, preferred_element_type=jnp.float32).astype(jnp.bfloat16)
    dc = (((0,), (0,)), ((), ()))
    p0 = lax.dot_general(vb, vb, dc, preferred_element_type=jnp.float32)
    p1 = lax.dot_general(w1, vb, dc, preferred_element_type=jnp.float32)
    p2 = lax.dot_general(w2, vb, dc, preferred_element_type=jnp.float32)
    p3 = lax.dot_general(w3, vb, dc, preferred_element_type=jnp.float32)
    part = jnp.concatenate([p0, p1, p2, p3], axis=0)

    @pl.when(i == 0)
    def _():
        out_ref[...] = jnp.zeros_like(out_ref)

    out_ref[...] += part


def _s2_call(fv, d4):
    return pl.pallas_call(
        _s2_body,
        grid=(_NBLK,),
        in_specs=[
            pl.BlockSpec((_BR, 128), lambda i: (i, 0)),
            pl.BlockSpec((_BR, _TPR), lambda i: (i, 0)),
        ],
        out_specs=pl.BlockSpec((4 * 128, 128), lambda i: (0, 0)),
        out_shape=jax.ShapeDtypeStruct((4 * 128, 128), jnp.float32),
        compiler_params=pltpu.CompilerParams(
            dimension_semantics=("arbitrary",),
        ),
    )(fv, d4)


def _final_body(sum_ref, cnt_ref, t_ref, out_ref):
    sums2 = sum_ref[...]
    sums = (sums2[:_SEG, :] + sums2[_SEG:, :]).reshape(_M, _C, _D)
    cnt32 = cnt_ref[...].T
    cnts = jnp.sum(cnt32, axis=1).reshape(_M, _C)
    t = t_ref[...]
    tt = []
    for h in range(4):
        acc = jnp.zeros((_D, _D), jnp.float32)
        for a in range(_TPR):
            acc = acc + t[128 * h + _D * a:128 * h + _D * (a + 1),
                          _D * a:_D * (a + 1)]
        tt.append(acc)
    t0, t1, t2, t3 = tt

    present = cnts > 0.0
    mu_dc = jnp.where(present[..., None],
                      sums / jnp.maximum(cnts, 1.0)[..., None], 0.0)
    anchors_dc = jnp.where(present[..., None], (1.0 - _MOM) * mu_dc, 0.0)

    anchor_global = jnp.zeros((_C, _D), jnp.float32)
    for d in range(_M):
        upd = _MOM * anchor_global + (1.0 - _MOM) * mu_dc[d]
        anchor_global = jnp.where(present[d][:, None], upd, anchor_global)
    counts_global = cnts.sum(axis=0)

    total = sums.sum(axis=0).sum(axis=0)
    nf = jnp.float32(_N)
    mu = total / nf
    cov = (t0 - nf * mu[:, None] * mu[None, :]) / (nf + 1e-06)
    gm = (1.0 - _MOM) * mu
    r_i = lax.broadcasted_iota(jnp.int32, (_D, _D), 0)
    c_i = lax.broadcasted_iota(jnp.int32, (_D, _D), 1)
    eye = (r_i == c_i).astype(jnp.float32)
    gc = _MOM * eye + (1.0 - _MOM) * cov

    valid = present & (counts_global > 0.0)[None, :]
    per = ((anchors_dc - anchor_global[None]) ** 2).mean(axis=-1)
    nvalid = valid.sum()
    caa = jnp.where(
        nvalid > 0,
        jnp.where(valid, per, 0.0).sum()
        / jnp.maximum(nvalid, 1).astype(jnp.float32),
        0.0,
    )

    loss_s = jnp.float32(0.0)
    vs = jnp.float32(0.0)
    for d in range(_M):
        n_d = cnts[d].sum()
        sum_d = sums[d].sum(axis=0)
        mu_d = sum_d / jnp.maximum(n_d, 1.0)
        s1 = 1.0 - 2.0 * (d & 1)
        s2 = 1.0 - 2.0 * (d >> 1)
        s2_d = 0.25 * (t0 + s1 * t1 + s2 * t2 + (s1 * s2) * t3)
        cov_d = (s2_d - n_d * mu_d[:, None] * mu_d[None, :]) / (n_d + 1e-06)
        term = ((mu_d - gm) ** 2).mean() + ((cov_d - gc) ** 2).mean()
        ok = (n_d > 0).astype(jnp.float32)
        loss_s = loss_s + ok * term
        vs = vs + ok
    stats = jnp.where(vs > 0, loss_s / jnp.maximum(vs, 1.0), 0.0)

    out_ref[...] = jnp.full((1, 1), caa + stats, jnp.float32)


def _final_call(sums, cnts, t):
    return pl.pallas_call(
        _final_body,
        out_shape=jax.ShapeDtypeStruct((1, 1), jnp.float32),
    )(sums, cnts, t)


def kernel(feats, labels, domain_ids):
    sums, cnts = _sc_partials(feats, labels, domain_ids)
    t = _s2_call(feats.reshape(_NR, 128),
                 domain_ids.reshape(_NR, _TPR))
    out = _final_call(sums.reshape(_NC * _SEG, _D), cnts, t)
    return out.reshape(())


# speedup vs baseline: 3.2048x; 1.0096x over previous
import functools

import jax
import jax.numpy as jnp
from jax import lax
from jax.experimental import pallas as pl
from jax.experimental.pallas import tpu as pltpu
from jax.experimental.pallas import tpu_sc as plsc

_C = 256
_D = 32
_M = 4
_N = 1000000
_MOM = 0.9

_NC = 2
_NS = 16
_L = 16
_NW = _NC * _NS
_SEG = _M * _C

_CH = 2000
_GPC = _CH // _L
_NCHTOT = _N // _CH
_CPT = (_NCHTOT + _NW - 1) // _NW
_ZR = 128


def _sc_body(ft_hbm, lab_hbm, dom_hbm, sum_hbm, cnt_hbm,
             feat_v, lab_v, dom_v, idx_v, cnt_v, zero_v, shared, sem):
    core = lax.axis_index("c")
    sid = lax.axis_index("s")
    wid = sid * _NC + core
    iota = lax.iota(jnp.int32, _L)
    zf = jnp.zeros((_L,), jnp.float32)
    ones = jnp.ones((_L,), jnp.float32)

    def zcnt(s, _):
        cnt_v[pl.ds(s * _L, _L)] = zf
        return ()

    lax.fori_loop(0, _SEG // _L, zcnt, (), unroll=8)

    @pl.when(sid == 0)
    def _():
        def zrow(r, _):
            zero_v[r, pl.ds(0, _L)] = zf
            zero_v[r, pl.ds(_L, _L)] = zf
            return ()

        lax.fori_loop(0, _ZR, zrow, (), unroll=8)

        def zshared(b, _):
            pltpu.sync_copy(zero_v, shared.at[pl.ds(b * _ZR, _ZR), :])
            return ()

        lax.fori_loop(0, _SEG // _ZR, zshared, ())

    plsc.subcore_barrier()

    def gbody(g, _):
        labv = lab_v[pl.ds(g * _L, _L)]
        domv = dom_v[pl.ds(g * _L, _L)]
        segv = domv * _C + labv
        idx_v[pl.ds(g * _L, _L)] = segv
        plsc.addupdate_scatter(cnt_v, [segv], ones)
        return ()

    def cbody(cl, _):
        cid = wid + _NW * cl

        @pl.when(cid < _NCHTOT)
        def _():
            c0 = cid * _CH
            pltpu.async_copy(ft_hbm.at[pl.ds(c0, _CH), :], feat_v,
                             sem).wait()
            pltpu.sync_copy(lab_hbm.at[pl.ds(c0, _CH)], lab_v)
            pltpu.sync_copy(dom_hbm.at[pl.ds(c0, _CH)], dom_v)
            lax.fori_loop(0, _GPC, gbody, ())
            pltpu.sync_copy(feat_v, shared.at[idx_v], add=True)

        return ()

    lax.fori_loop(0, _CPT, cbody, ())
    plsc.subcore_barrier()

    @pl.when(sid == 0)
    def _():
        pltpu.sync_copy(shared, sum_hbm.at[core])

    pltpu.sync_copy(cnt_v, cnt_hbm.at[wid])


_sc_partials = functools.partial(
    pl.kernel,
    out_type=(
        jax.ShapeDtypeStruct((_NC, _SEG, _D), jnp.float32),
        jax.ShapeDtypeStruct((_NW, _SEG), jnp.float32),
    ),
    mesh=plsc.VectorSubcoreMesh(core_axis_name="c", subcore_axis_name="s"),
    compiler_params=pltpu.CompilerParams(
        needs_layout_passes=False,
        use_tc_tiling_on_sc=False,
    ),
    scratch_types=[
        pltpu.VMEM((_CH, _D), jnp.float32),
        pltpu.VMEM((_CH,), jnp.int32),
        pltpu.VMEM((_CH,), jnp.int32),
        pltpu.VMEM((_CH,), jnp.int32),
        pltpu.VMEM((_SEG,), jnp.float32),
        pltpu.VMEM((_ZR, _D), jnp.float32),
        pltpu.VMEM_SHARED((_SEG, _D), jnp.float32),
        pltpu.SemaphoreType.DMA,
    ],
)(_sc_body)


_TPR = 128 // _D
_NR = _N // _TPR
_BR = 5000
_NBLK = _NR // _BR


def _s2_body(fv_ref, d4_ref, out_ref):
    i = pl.program_id(0)
    vb = fv_ref[...].astype(jnp.bfloat16)
    d4 = d4_ref[...]
    h1 = (1 - 2 * (d4 & 1)).astype(jnp.bfloat16)
    h2 = (1 - 2 * (d4 >> 1)).astype(jnp.bfloat16)
    h3 = h1 * h2
    r_i = lax.broadcasted_iota(jnp.int32, (_TPR, 128), 0)
    c_i = lax.broadcasted_iota(jnp.int32, (_TPR, 128), 1)
    rep = (c_i // _D == r_i).astype(jnp.bfloat16)
    dn = (((1,), (0,)), ((), ()))
    w1 = vb * lax.dot_general(
        h1, rep, dn, preferred_element_type=jnp.float32).astype(jnp.bfloat16)
    w2 = vb * lax.dot_general(
        h2, rep, dn, preferred_element_type=jnp.float32).astype(jnp.bfloat16)
    w3 = vb * lax.dot_general(
        h3, rep, dn, preferred_element_type=jnp.float32).astype(jnp.bfloat16)
    dc = (((0,), (0,)), ((), ()))
    p0 = lax.dot_general(vb, vb, dc, preferred_element_type=jnp.float32)
    p1 = lax.dot_general(w1, vb, dc, preferred_element_type=jnp.float32)
    p2 = lax.dot_general(w2, vb, dc, preferred_element_type=jnp.float32)
    p3 = lax.dot_general(w3, vb, dc, preferred_element_type=jnp.float32)
    part = jnp.concatenate([p0, p1, p2, p3], axis=0)

    @pl.when(i == 0)
    def _():
        out_ref[...] = jnp.zeros_like(out_ref)

    out_ref[...] += part


def _s2_call(fv, d4):
    return pl.pallas_call(
        _s2_body,
        grid=(_NBLK,),
        in_specs=[
            pl.BlockSpec((_BR, 128), lambda i: (i, 0)),
            pl.BlockSpec((_BR, _TPR), lambda i: (i, 0)),
        ],
        out_specs=pl.BlockSpec((4 * 128, 128), lambda i: (0, 0)),
        out_shape=jax.ShapeDtypeStruct((4 * 128, 128), jnp.float32),
        compiler_params=pltpu.CompilerParams(
            dimension_semantics=("arbitrary",),
        ),
    )(fv, d4)


def _final_body(sum_ref, cnt_ref, t_ref, out_ref):
    sums2 = sum_ref[...]
    sums = (sums2[:_SEG, :] + sums2[_SEG:, :]).reshape(_M, _C, _D)
    cnt32 = cnt_ref[...].T
    cnts = jnp.sum(cnt32, axis=1).reshape(_M, _C)
    t = t_ref[...]
    tt = []
    for h in range(4):
        acc = jnp.zeros((_D, _D), jnp.float32)
        for a in range(_TPR):
            acc = acc + t[128 * h + _D * a:128 * h + _D * (a + 1),
                          _D * a:_D * (a + 1)]
        tt.append(acc)
    t0, t1, t2, t3 = tt

    present = cnts > 0.0
    mu_dc = jnp.where(present[..., None],
                      sums / jnp.maximum(cnts, 1.0)[..., None], 0.0)
    anchors_dc = jnp.where(present[..., None], (1.0 - _MOM) * mu_dc, 0.0)

    anchor_global = jnp.zeros((_C, _D), jnp.float32)
    for d in range(_M):
        upd = _MOM * anchor_global + (1.0 - _MOM) * mu_dc[d]
        anchor_global = jnp.where(present[d][:, None], upd, anchor_global)
    counts_global = cnts.sum(axis=0)

    total = sums.sum(axis=0).sum(axis=0)
    nf = jnp.float32(_N)
    mu = total / nf
    cov = (t0 - nf * mu[:, None] * mu[None, :]) / (nf + 1e-06)
    gm = (1.0 - _MOM) * mu
    r_i = lax.broadcasted_iota(jnp.int32, (_D, _D), 0)
    c_i = lax.broadcasted_iota(jnp.int32, (_D, _D), 1)
    eye = (r_i == c_i).astype(jnp.float32)
    gc = _MOM * eye + (1.0 - _MOM) * cov

    valid = present & (counts_global > 0.0)[None, :]
    per = ((anchors_dc - anchor_global[None]) ** 2).mean(axis=-1)
    nvalid = valid.sum()
    caa = jnp.where(
        nvalid > 0,
        jnp.where(valid, per, 0.0).sum()
        / jnp.maximum(nvalid, 1).astype(jnp.float32),
        0.0,
    )

    loss_s = jnp.float32(0.0)
    vs = jnp.float32(0.0)
    for d in range(_M):
        n_d = cnts[d].sum()
        sum_d = sums[d].sum(axis=0)
        mu_d = sum_d / jnp.maximum(n_d, 1.0)
        s1 = 1.0 - 2.0 * (d & 1)
        s2 = 1.0 - 2.0 * (d >> 1)
        s2_d = 0.25 * (t0 + s1 * t1 + s2 * t2 + (s1 * s2) * t3)
        cov_d = (s2_d - n_d * mu_d[:, None] * mu_d[None, :]) / (n_d + 1e-06)
        term = ((mu_d - gm) ** 2).mean() + ((cov_d - gc) ** 2).mean()
        ok = (n_d > 0).astype(jnp.float32)
        loss_s = loss_s + ok * term
        vs = vs + ok
    stats = jnp.where(vs > 0, loss_s / jnp.maximum(vs, 1.0), 0.0)

    out_ref[...] = jnp.full((1, 1), caa + stats, jnp.float32)


def _final_call(sums, cnts, t):
    return pl.pallas_call(
        _final_body,
        out_shape=jax.ShapeDtypeStruct((1, 1), jnp.float32),
    )(sums, cnts, t)


def kernel(feats, labels, domain_ids):
    sums, cnts = _sc_partials(feats, labels, domain_ids)
    t = _s2_call(feats.reshape(_NR, 128),
                 domain_ids.reshape(_NR, _TPR))
    out = _final_call(sums.reshape(_NC * _SEG, _D), cnts, t)
    return out.reshape(())
